# Initial kernel scaffold; baseline (speedup 1.0000x reference)
import jax
import jax.numpy as jnp
from jax.experimental import pallas as pl


def kernel(x, edge_index, W_rel, W_root, b):
    raise NotImplementedError("write your pallas kernel here")


# speedup vs baseline: 3.0016x; 3.0016x over previous
import functools

import jax
import jax.numpy as jnp
from jax import lax
from jax.experimental import pallas as pl
from jax.experimental.pallas import tpu as pltpu
from jax.experimental.pallas import tpu_sc as plsc

_CHUNK = 128
_NC = 2
_NS = 16


def _make_sc_agg(N_acc, D, n_chunks):
    mesh = plsc.VectorSubcoreMesh(core_axis_name="c", subcore_axis_name="s")
    rows_tile = N_acc // _NS

    @functools.partial(
        pl.kernel,
        out_type=jax.ShapeDtypeStruct((_NC, N_acc, D), jnp.float32),
        mesh=mesh,
        scratch_types=[
            pltpu.VMEM_SHARED((N_acc, D), jnp.float32),
            pltpu.VMEM((n_chunks // 2, _CHUNK), jnp.int32),
            pltpu.VMEM((n_chunks // 2, _CHUNK), jnp.int32),
            pltpu.VMEM((_CHUNK, D), jnp.float32),
            pltpu.VMEM((_CHUNK, D), jnp.float32),
            pltpu.SemaphoreType.DMA,
            pltpu.SemaphoreType.DMA,
        ],
    )
    def sc_agg(x_hbm, src_hbm, dst_hbm, zero_hbm, out_hbm,
               acc_sh, src_v, dst_v, buf0, buf1, sem0, sem1):
        cid = lax.axis_index("c")
        sid = lax.axis_index("s")
        pltpu.sync_copy(zero_hbm.at[pl.ds(sid * rows_tile, rows_tile)],
                        acc_sh.at[pl.ds(sid * rows_tile, rows_tile)])
        plsc.subcore_barrier()

        wbase = (cid * _NS + sid) * n_chunks
        n_half = n_chunks // 2
        bufs = (buf0, buf1)
        sems = (sem0, sem1)
        for half in range(2):
            hbase = wbase + half * n_half
            pltpu.sync_copy(src_hbm.at[pl.ds(hbase, n_half)], src_v)
            pltpu.sync_copy(dst_hbm.at[pl.ds(hbase, n_half)], dst_v)
            pltpu.async_copy(x_hbm.at[src_v.at[0]], buf0, sem0)
            pltpu.async_copy(x_hbm.at[src_v.at[1]], buf1, sem1)

            def step(j, carry):
                for bi in range(2):
                    jj = j + bi
                    buf, sem = bufs[bi], sems[bi]
                    pltpu.make_async_copy(
                        x_hbm.at[src_v.at[jj]], buf, sem).wait()
                    pltpu.sync_copy(buf, acc_sh.at[dst_v.at[jj]], add=True)

                    @pl.when(jj + 2 < n_half)
                    def _start_next():
                        pltpu.async_copy(
                            x_hbm.at[src_v.at[jj + 2]], buf, sem)
                return carry

            lax.fori_loop(0, n_half // 2, lambda i, c: step(2 * i, c), 0)
        plsc.subcore_barrier()
        pltpu.sync_copy(acc_sh.at[pl.ds(sid * rows_tile, rows_tile)],
                        out_hbm.at[cid, pl.ds(sid * rows_tile, rows_tile)])

    return sc_agg


def _make_tc_combine(N, D, BN, relu, res):
    def body(agg0, agg1, x, wr, ws, bb, o):
        s = agg0[...] + agg1[...]
        acc = jnp.dot(s, wr[...], preferred_element_type=jnp.float32)
        acc = acc + jnp.dot(x[...], ws[...], preferred_element_type=jnp.float32)
        acc = acc + bb[...]
        if relu:
            acc = jnp.maximum(acc, 0.0)
        if res:
            acc = acc + x[...]
        o[...] = acc

    row_spec = pl.BlockSpec((BN, D), lambda i: (i, 0))
    full_spec = pl.BlockSpec((D, D), lambda i: (0, 0))
    return pl.pallas_call(
        body,
        grid=(N // BN,),
        in_specs=[row_spec, row_spec, row_spec, full_spec, full_spec,
                  pl.BlockSpec((1, D), lambda i: (0, 0))],
        out_specs=row_spec,
        out_shape=jax.ShapeDtypeStruct((N, D), jnp.float32),
    )


def kernel(x, edge_index, W_rel, W_root, b):
    N, D = x.shape
    L = W_rel.shape[0]
    E = edge_index.shape[1]
    NW = _NC * _NS

    per_w_chunks = -(-E // (_CHUNK * NW * 2)) * 2
    E_pad = per_w_chunks * _CHUNK * NW
    N_acc = -(-(N + 1) // (_NS * 8)) * (_NS * 8)
    pad = E_pad - E
    src = jnp.concatenate(
        [edge_index[0], jnp.zeros((pad,), jnp.int32)]).reshape(-1, _CHUNK)
    dst = jnp.concatenate(
        [edge_index[1], jnp.full((pad,), N, jnp.int32)]).reshape(-1, _CHUNK)
    zero = jnp.zeros((N_acc, D), jnp.float32)

    Wr_t = jnp.swapaxes(W_rel, 1, 2)
    Ws_t = jnp.swapaxes(W_root, 1, 2)

    sc_agg = _make_sc_agg(N_acc, D, per_w_chunks)
    BN = 1000 if N % 1000 == 0 else 8
    combines = {}

    h = x
    for i in range(L):
        aggs = sc_agg(h, src, dst, zero)
        key = (i < L - 1, i > 0)
        if key not in combines:
            combines[key] = _make_tc_combine(N, D, BN, *key)
        h = combines[key](aggs[0, :N], aggs[1, :N], h,
                          Wr_t[i], Ws_t[i], b[i][None])
    return h
